# baseline (speedup 1.0000x reference)
import functools

import jax
import jax.numpy as jnp
from jax.experimental import pallas as pl
from jax.experimental.pallas import tpu as pltpu
from jax.experimental.pallas import tpu_sc as plsc

N = 10000
K = 30
DH = 128
NUM_RBF = 16
E = N * K
RB = 200
EB = RB * K
NBLK = N // RB
HN = N // 2
HE = HN * K
HBLK = NBLK // 2
HEPAD = 151552


_LOG2E = 1.4426950408889634
_LN2 = 0.6931471805599453


def _softplus(x):
    return jnp.maximum(x, 0.0) + _LN2 * jnp.log2(
        1.0 + jnp.exp2(-jnp.abs(x) * _LOG2E))


def _ln(x):
    mu = jnp.mean(x, axis=-1, keepdims=True)
    var = jnp.mean((x - mu) ** 2, axis=-1, keepdims=True)
    return (x - mu) / jnp.sqrt(var + 1e-5)


NP = 10240
NC = NP // 128
DEPTH = 8


def _knn_embed_body(x2_ref, x2tc_ref, wn_ref, bn_ref,
                    tab_ref, d2sel_ref, eidx_ref, vstk_ref, cstk_ref):
    i = pl.program_id(0)
    x2 = x2_ref[...]

    xr = (x2[:, 0:1] + x2[:, 3:4] + x2[:, 6:7] + x2[:, 9:10]) * 0.25
    yr = (x2[:, 1:2] + x2[:, 4:5] + x2[:, 7:8] + x2[:, 10:11]) * 0.25
    zr = (x2[:, 2:3] + x2[:, 5:6] + x2[:, 8:9] + x2[:, 11:12]) * 0.25

    row_ids = i * RB + jax.lax.broadcasted_iota(jnp.int32, (RB, 1), 0)
    lane = jax.lax.broadcasted_iota(jnp.int32, (1, 128), 1)

    for t in range(DEPTH):
        vstk_ref[t] = jnp.full((RB, 128), jnp.inf, jnp.float32)
        cstk_ref[t] = jnp.full((RB, 128), 2 ** 30, jnp.int32)

    def chunk_body(c, _):
        xt2 = x2tc_ref[c]
        xT = (xt2[0:1] + xt2[3:4] + xt2[6:7] + xt2[9:10]) * 0.25
        yT = (xt2[1:2] + xt2[4:5] + xt2[7:8] + xt2[10:11]) * 0.25
        zT = (xt2[2:3] + xt2[5:6] + xt2[8:9] + xt2[11:12]) * 0.25
        dx = xr - xT
        dy = yr - yT
        dz = zr - zT
        d2c = dx * dx + dy * dy + dz * dz
        colnum = c * 128 + lane
        d2c = jnp.where(row_ids == colnum, 1e9, d2c)
        v = d2c
        cc = jnp.broadcast_to(colnum, (RB, 128))
        for t in range(DEPTH):
            vt = vstk_ref[t]
            ct = cstk_ref[t]
            swap = v < vt
            vstk_ref[t] = jnp.where(swap, v, vt)
            cstk_ref[t] = jnp.where(swap, cc, ct)
            v = jnp.where(swap, vt, v)
            cc = jnp.where(swap, ct, cc)
        return 0

    jax.lax.fori_loop(0, NC, chunk_body, 0)

    lane_k = jax.lax.broadcasted_iota(jnp.int32, (1, K), 1)

    def body(k, carry):
        vals, idxs = carry
        vall = vstk_ref[...]
        call = cstk_ref[...]
        m2 = jnp.min(vall, axis=0)
        m = jnp.min(m2, axis=1, keepdims=True)
        cand = jnp.where(vall == m[None], call, jnp.int32(2 ** 30))
        s2 = jnp.min(cand, axis=0)
        sel = jnp.min(s2, axis=1, keepdims=True)
        vstk_ref[...] = jnp.where(call == sel[None], jnp.inf, vall)
        vals = jnp.where(lane_k == k, m, vals)
        idxs = jnp.where(lane_k == k, sel, idxs)
        return vals, idxs

    vals0 = jnp.zeros((RB, K), jnp.float32)
    idxs0 = jnp.zeros((RB, K), jnp.int32)
    vals, idxs = jax.lax.fori_loop(0, K, body, (vals0, idxs0))
    d2sel_ref[...] = vals
    eidx_ref[...] = idxs

    xc3 = jnp.concatenate([xr, yr, zr], axis=1)
    xrel = x2 - jnp.concatenate([xc3, xc3, xc3, xc3], axis=1)
    bond = x2[:, 3:12] - x2[:, 0:9]
    lls = []
    for t in range(3):
        b = bond[:, 3 * t:3 * t + 3]
        sq = jnp.sum(b * b, axis=1, keepdims=True)
        lls.append(jnp.log(jnp.sqrt(sq + 1e-8) + 1e-6))
    feat = jnp.concatenate([xrel] + lls, axis=1)
    nh = (jnp.dot(feat, wn_ref[...],
                  preferred_element_type=jnp.float32) + bn_ref[...])
    tab_ref[...] = jnp.concatenate(
        [nh, xc3, jnp.zeros((RB, 125), jnp.float32)], axis=1)


def _knn_embed(x2, x2tc, wn, bn):
    return pl.pallas_call(
        _knn_embed_body,
        grid=(NBLK,),
        in_specs=[
            pl.BlockSpec((RB, 12), lambda i: (i, 0)),
            pl.BlockSpec((NC, 12, 128), lambda i: (0, 0, 0)),
            pl.BlockSpec((15, DH), lambda i: (0, 0)),
            pl.BlockSpec((1, DH), lambda i: (0, 0)),
        ],
        out_specs=[
            pl.BlockSpec((RB, 2 * DH), lambda i: (i, 0)),
            pl.BlockSpec((RB, K), lambda i: (i, 0)),
            pl.BlockSpec((RB, K), lambda i: (i, 0)),
        ],
        out_shape=[
            jax.ShapeDtypeStruct((N, 2 * DH), jnp.float32),
            jax.ShapeDtypeStruct((N, K), jnp.float32),
            jax.ShapeDtypeStruct((N, K), jnp.int32),
        ],
        scratch_shapes=[pltpu.VMEM((DEPTH, RB, 128), jnp.float32),
                        pltpu.VMEM((DEPTH, RB, 128), jnp.int32)],
    )(x2, x2tc, wn, bn)


_GW = 128


def _gather_rows(table, idx_pad):
    v = table.shape[1]
    m = idx_pad.shape[1]

    @pl.kernel(
        out_type=jax.ShapeDtypeStruct((m, v), table.dtype),
        mesh=plsc.VectorSubcoreMesh(core_axis_name="core",
                                    subcore_axis_name="subcore"),
    )
    def k(x_hbm, i_hbm, o_hbm):
        def body(i_vmem, o_vmem):
            pltpu.sync_copy(x_hbm.at[i_vmem.at[0]], o_vmem)

        pltpu.emit_pipeline(
            body,
            grid=(m // _GW,),
            in_specs=[pl.BlockSpec((1, _GW), lambda i: (0, i))],
            out_specs=[pl.BlockSpec((_GW, v), lambda i: (i, 0))],
            core_axis_name=("core", "subcore"),
            dimension_semantics=(pltpu.PARALLEL,),
        )(i_hbm, o_hbm)

    return k(table, idx_pad)


def _edge_feat_body(d2_ref, xc_ref, xcj_ref, we_ref, be_ref, eh_ref):
    d2b = d2_ref[...]
    dd = jnp.sqrt(d2b + 1e-8)
    xci = xc_ref[:, 0:3]
    xci = jnp.broadcast_to(xci[:, None, :], (RB, K, 3)).reshape(EB, 3)
    xcj = xcj_ref[...]
    cen = jax.lax.broadcasted_iota(
        jnp.int32, (1, NUM_RBF), 1).astype(jnp.float32) * (20.0 / 15.0)
    sigma = 20.0 / NUM_RBF
    rbf = jnp.exp2(-(((dd - cen) / sigma) ** 2) * _LOG2E)
    dirv = (xcj[:, 0:3] - xci) / (dd + 1e-8)
    ones = jnp.ones((EB, 1), jnp.float32)
    feat = jnp.concatenate([rbf, dirv, ones], axis=1)
    eh_ref[...] = (jnp.dot(feat, we_ref[...],
                           preferred_element_type=jnp.float32) + be_ref[...])


def _edge_feat(d2flat, table0, xcj, we, be, off):
    return pl.pallas_call(
        _edge_feat_body,
        grid=(HBLK,),
        in_specs=[
            pl.BlockSpec((EB, 1), lambda i: (i, 0)),
            pl.BlockSpec((RB, DH), lambda i: (i + off * HBLK, 1)),
            pl.BlockSpec((EB, DH), lambda i: (i, 1)),
            pl.BlockSpec((NUM_RBF + 4, DH), lambda i: (0, 0)),
            pl.BlockSpec((1, DH), lambda i: (0, 0)),
        ],
        out_specs=pl.BlockSpec((EB, DH), lambda i: (i, 0)),
        out_shape=jax.ShapeDtypeStruct((HE, DH), jnp.float32),
    )(d2flat, table0, xcj, we, be)


def _msg_node_body(nh_ref, hj_ref, eh_ref, wm1_ref, bm1_ref, wm2_ref, bm2_ref,
                   wd1_ref, bd1_ref, wd2_ref, bd2_ref, buf_ref, out_ref):
    del buf_ref
    hi = nh_ref[...]
    w1a = wm1_ref[0:DH, :]
    w1b = wm1_ref[DH:2 * DH, :]
    w1c = wm1_ref[2 * DH:3 * DH, :]
    t1 = jnp.dot(hi, w1a, preferred_element_type=jnp.float32) + bm1_ref[...]
    t1r = jnp.broadcast_to(t1[:, None, :], (RB, K, DH)).reshape(EB, DH)
    z = (t1r
         + jnp.dot(hj_ref[...], w1b, preferred_element_type=jnp.float32)
         + jnp.dot(eh_ref[...], w1c, preferred_element_type=jnp.float32))
    m = (jnp.dot(_softplus(z), wm2_ref[...],
                 preferred_element_type=jnp.float32) + bm2_ref[...])
    agg = jnp.sum(m.reshape(RB, K, DH), axis=1) / float(K)
    h = _ln(hi + agg)
    dh = (jnp.dot(_softplus(jnp.dot(h, wd1_ref[...],
                                    preferred_element_type=jnp.float32)
                            + bd1_ref[...]),
                  wd2_ref[...], preferred_element_type=jnp.float32)
          + bd2_ref[...])
    out_ref[...] = _ln(h + dh)


def _msg_node(nh, hj, eh, wm1, bm1, wm2, bm2, wd1, bd1, wd2, bd2, buf, off):
    return pl.pallas_call(
        _msg_node_body,
        grid=(HBLK,),
        in_specs=[
            pl.BlockSpec((RB, DH), lambda i, o=off: (i + o, 0)),
            pl.BlockSpec((EB, DH), lambda i: (i, 0)),
            pl.BlockSpec((EB, DH), lambda i: (i, 0)),
            pl.BlockSpec((3 * DH, DH), lambda i: (0, 0)),
            pl.BlockSpec((1, DH), lambda i: (0, 0)),
            pl.BlockSpec((DH, DH), lambda i: (0, 0)),
            pl.BlockSpec((1, DH), lambda i: (0, 0)),
            pl.BlockSpec((DH, DH), lambda i: (0, 0)),
            pl.BlockSpec((1, DH), lambda i: (0, 0)),
            pl.BlockSpec((DH, DH), lambda i: (0, 0)),
            pl.BlockSpec((1, DH), lambda i: (0, 0)),
            pl.BlockSpec((RB, DH), lambda i, o=off: (i + o, 0)),
        ],
        out_specs=pl.BlockSpec((RB, DH), lambda i, o=off: (i + o, 0)),
        out_shape=jax.ShapeDtypeStruct((N, DH), jnp.float32),
        input_output_aliases={11: 0},
    )(nh, hj, eh, wm1, bm1, wm2, bm2, wd1, bd1, wd2, bd2, buf)


def _edge_upd_body(nh_ref, hj_ref, eh_ref, we1_ref, be1_ref, we2_ref, be2_ref,
                   out_ref):
    hi = nh_ref[...]
    w1a = we1_ref[0:DH, :]
    w1b = we1_ref[DH:2 * DH, :]
    w1c = we1_ref[2 * DH:3 * DH, :]
    t1 = jnp.dot(hi, w1a, preferred_element_type=jnp.float32) + be1_ref[...]
    t1r = jnp.broadcast_to(t1[:, None, :], (RB, K, DH)).reshape(EB, DH)
    eh = eh_ref[...]
    z = (t1r
         + jnp.dot(hj_ref[...], w1b, preferred_element_type=jnp.float32)
         + jnp.dot(eh, w1c, preferred_element_type=jnp.float32))
    de = (jnp.dot(_softplus(z), we2_ref[...],
                  preferred_element_type=jnp.float32) + be2_ref[...])
    out_ref[...] = _ln(eh + de)


def _edge_upd_final_body(nh_ref, hj_ref, eh_ref, we1_ref, be1_ref, we2_ref,
                         be2_ref, buf_ref, out_ref):
    del buf_ref
    _edge_upd_body(nh_ref, hj_ref, eh_ref, we1_ref, be1_ref, we2_ref,
                   be2_ref, out_ref)


def _edge_upd_final(nh, hj, eh, we1, be1, we2, be2, buf, off):
    return pl.pallas_call(
        _edge_upd_final_body,
        grid=(HBLK,),
        in_specs=[
            pl.BlockSpec((RB, DH), lambda i, o=off: (i + o, 0)),
            pl.BlockSpec((EB, DH), lambda i: (i, 0)),
            pl.BlockSpec((EB, DH), lambda i: (i, 0)),
            pl.BlockSpec((3 * DH, DH), lambda i: (0, 0)),
            pl.BlockSpec((1, DH), lambda i: (0, 0)),
            pl.BlockSpec((DH, DH), lambda i: (0, 0)),
            pl.BlockSpec((1, DH), lambda i: (0, 0)),
            pl.BlockSpec((EB, DH), lambda i, o=off: (i + o, 0)),
        ],
        out_specs=pl.BlockSpec((EB, DH), lambda i, o=off: (i + o, 0)),
        out_shape=jax.ShapeDtypeStruct((E, DH), jnp.float32),
        input_output_aliases={7: 0},
    )(nh, hj, eh, we1, be1, we2, be2, buf)


def _edge_upd(nh, hj, eh, we1, be1, we2, be2, off):
    return pl.pallas_call(
        _edge_upd_body,
        grid=(HBLK,),
        in_specs=[
            pl.BlockSpec((RB, DH), lambda i, o=off: (i + o, 0)),
            pl.BlockSpec((EB, DH), lambda i: (i, 0)),
            pl.BlockSpec((EB, DH), lambda i: (i, 0)),
            pl.BlockSpec((3 * DH, DH), lambda i: (0, 0)),
            pl.BlockSpec((1, DH), lambda i: (0, 0)),
            pl.BlockSpec((DH, DH), lambda i: (0, 0)),
            pl.BlockSpec((1, DH), lambda i: (0, 0)),
        ],
        out_specs=pl.BlockSpec((EB, DH), lambda i: (i, 0)),
        out_shape=jax.ShapeDtypeStruct((HE, DH), jnp.float32),
    )(nh, hj, eh, we1, be1, we2, be2)


def _zeros(shape):
    return jnp.zeros(shape, jnp.float32)


def kernel(X, C, W_node, b_node, W_edge, b_edge, Wm1, bm1, Wm2, bm2,
           Wd1, bd1, Wd2, bd2, We1, be1, We2, be2):
    B = X.shape[0]
    x2 = X.reshape(N, 12)
    x2t_pad = jnp.concatenate(
        [x2.T, jnp.full((12, NP - N), 1e6, jnp.float32)], axis=1)
    x2tc = x2t_pad.reshape(12, NC, 128).transpose(1, 0, 2)
    table0, d2sel, eidx = _knn_embed(
        x2, x2tc, W_node, b_node.reshape(1, DH))

    def pad_idx(half):
        return jnp.concatenate(
            [half.reshape(1, HE), jnp.zeros((1, HEPAD - HE), jnp.int32)],
            axis=1)

    idxA = pad_idx(eidx[:HN])
    idxB = pad_idx(eidx[HN:])

    gA = _gather_rows(table0, idxA)
    gB = _gather_rows(table0, idxB)
    be_ = b_edge.reshape(1, DH)
    ehA = _edge_feat(d2sel[:HN].reshape(HE, 1), table0, gA, W_edge, be_, 0)
    ehB = _edge_feat(d2sel[HN:].reshape(HE, 1), table0, gB, W_edge, be_, 1)

    nh_src = table0
    hjA, hjB = gA, gB
    for l in range(3):
        wl = (Wm1[l], bm1[l].reshape(1, DH), Wm2[l], bm2[l].reshape(1, DH),
              Wd1[l], bd1[l].reshape(1, DH), Wd2[l], bd2[l].reshape(1, DH))
        nh_half = _msg_node(nh_src, hjA, ehA, *wl, _zeros((N, DH)), 0)
        nh_full = _msg_node(nh_src, hjB, ehB, *wl, nh_half, HBLK)
        hjA = _gather_rows(nh_full, idxA)
        hjB = _gather_rows(nh_full, idxB)
        el = (We1[l], be1[l].reshape(1, DH), We2[l], be2[l].reshape(1, DH))
        if l < 2:
            ehA = _edge_upd(nh_full, hjA, ehA, *el, 0)
            ehB = _edge_upd(nh_full, hjB, ehB, *el, HBLK)
        else:
            eh = _edge_upd_final(nh_full, hjA, ehA, *el, _zeros((E, DH)), 0)
            eh = _edge_upd_final(nh_full, hjB, ehB, *el, eh, HBLK)
        nh_src = nh_full

    mask_i = (C > 0).astype(jnp.float32)
    mask_ij = jnp.broadcast_to(mask_i[:, :, None], (B, N, K))
    return (nh_full.reshape(B, N, DH),
            eh.reshape(B, N, K, DH),
            eidx.reshape(B, N, K),
            mask_i,
            mask_ij)


# speedup vs baseline: 1.0178x; 1.0178x over previous
import functools

import jax
import jax.numpy as jnp
from jax.experimental import pallas as pl
from jax.experimental.pallas import tpu as pltpu
from jax.experimental.pallas import tpu_sc as plsc

N = 10000
K = 30
DH = 128
NUM_RBF = 16
E = N * K
RB = 200
EB = RB * K
NBLK = N // RB
HN = N // 2
HE = HN * K
HBLK = NBLK // 2
HEPAD = 151552


_LOG2E = 1.4426950408889634
_LN2 = 0.6931471805599453


def _softplus(x):
    return jnp.maximum(x, 0.0) + _LN2 * jnp.log2(
        1.0 + jnp.exp2(-jnp.abs(x) * _LOG2E))


def _ln(x):
    mu = jnp.mean(x, axis=-1, keepdims=True)
    var = jnp.mean((x - mu) ** 2, axis=-1, keepdims=True)
    return (x - mu) / jnp.sqrt(var + 1e-5)


NP = 10240
NC = NP // 128
DEPTH = 8


def _knn_embed_body(x2_ref, x2tc_ref, wn_ref, bn_ref,
                    tab_ref, d2sel_ref, eidx_ref, vstk_ref, cstk_ref):
    i = pl.program_id(0)
    x2 = x2_ref[...]

    xr = (x2[:, 0:1] + x2[:, 3:4] + x2[:, 6:7] + x2[:, 9:10]) * 0.25
    yr = (x2[:, 1:2] + x2[:, 4:5] + x2[:, 7:8] + x2[:, 10:11]) * 0.25
    zr = (x2[:, 2:3] + x2[:, 5:6] + x2[:, 8:9] + x2[:, 11:12]) * 0.25

    row_ids = i * RB + jax.lax.broadcasted_iota(jnp.int32, (RB, 1), 0)
    lane = jax.lax.broadcasted_iota(jnp.int32, (1, 128), 1)

    for t in range(DEPTH):
        vstk_ref[t] = jnp.full((RB, 128), jnp.inf, jnp.float32)
        cstk_ref[t] = jnp.full((RB, 128), 2 ** 30, jnp.int32)

    def chunk_body(c, _):
        xt2 = x2tc_ref[c]
        xT = (xt2[0:1] + xt2[3:4] + xt2[6:7] + xt2[9:10]) * 0.25
        yT = (xt2[1:2] + xt2[4:5] + xt2[7:8] + xt2[10:11]) * 0.25
        zT = (xt2[2:3] + xt2[5:6] + xt2[8:9] + xt2[11:12]) * 0.25
        dx = xr - xT
        dy = yr - yT
        dz = zr - zT
        d2c = dx * dx + dy * dy + dz * dz
        colnum = c * 128 + lane
        d2c = jnp.where(row_ids == colnum, 1e9, d2c)
        v = d2c
        cc = jnp.broadcast_to(colnum, (RB, 128))
        for t in range(DEPTH):
            vt = vstk_ref[t]
            ct = cstk_ref[t]
            swap = v < vt
            vstk_ref[t] = jnp.where(swap, v, vt)
            cstk_ref[t] = jnp.where(swap, cc, ct)
            v = jnp.where(swap, vt, v)
            cc = jnp.where(swap, ct, cc)
        return 0

    jax.lax.fori_loop(0, NC, chunk_body, 0)

    lane_k = jax.lax.broadcasted_iota(jnp.int32, (1, K), 1)

    def body(k, carry):
        vals, idxs = carry
        vall = vstk_ref[...]
        call = cstk_ref[...]
        m2 = jnp.min(vall, axis=0)
        m = jnp.min(m2, axis=1, keepdims=True)
        cand = jnp.where(vall == m[None], call, jnp.int32(2 ** 30))
        s2 = jnp.min(cand, axis=0)
        sel = jnp.min(s2, axis=1, keepdims=True)
        vstk_ref[...] = jnp.where(call == sel[None], jnp.inf, vall)
        vals = jnp.where(lane_k == k, m, vals)
        idxs = jnp.where(lane_k == k, sel, idxs)
        return vals, idxs

    vals0 = jnp.zeros((RB, K), jnp.float32)
    idxs0 = jnp.zeros((RB, K), jnp.int32)
    vals, idxs = jax.lax.fori_loop(0, K, body, (vals0, idxs0))
    d2sel_ref[...] = vals
    eidx_ref[...] = idxs

    xc3 = jnp.concatenate([xr, yr, zr], axis=1)
    xrel = x2 - jnp.concatenate([xc3, xc3, xc3, xc3], axis=1)
    bond = x2[:, 3:12] - x2[:, 0:9]
    lls = []
    for t in range(3):
        b = bond[:, 3 * t:3 * t + 3]
        sq = jnp.sum(b * b, axis=1, keepdims=True)
        lls.append(jnp.log(jnp.sqrt(sq + 1e-8) + 1e-6))
    feat = jnp.concatenate([xrel] + lls, axis=1)
    nh = (jnp.dot(feat, wn_ref[...],
                  preferred_element_type=jnp.float32) + bn_ref[...])
    tab_ref[...] = jnp.concatenate(
        [nh, xc3, jnp.zeros((RB, 125), jnp.float32)], axis=1)


def _knn_embed(x2, x2tc, wn, bn):
    return pl.pallas_call(
        _knn_embed_body,
        grid=(NBLK,),
        in_specs=[
            pl.BlockSpec((RB, 12), lambda i: (i, 0)),
            pl.BlockSpec((NC, 12, 128), lambda i: (0, 0, 0)),
            pl.BlockSpec((15, DH), lambda i: (0, 0)),
            pl.BlockSpec((1, DH), lambda i: (0, 0)),
        ],
        out_specs=[
            pl.BlockSpec((RB, 2 * DH), lambda i: (i, 0)),
            pl.BlockSpec((RB, K), lambda i: (i, 0)),
            pl.BlockSpec((RB, K), lambda i: (i, 0)),
        ],
        out_shape=[
            jax.ShapeDtypeStruct((N, 2 * DH), jnp.float32),
            jax.ShapeDtypeStruct((N, K), jnp.float32),
            jax.ShapeDtypeStruct((N, K), jnp.int32),
        ],
        scratch_shapes=[pltpu.VMEM((DEPTH, RB, 128), jnp.float32),
                        pltpu.VMEM((DEPTH, RB, 128), jnp.int32)],
    )(x2, x2tc, wn, bn)


_GW = 128


def _gather_rows(table, idx_pad):
    v = table.shape[1]
    m = idx_pad.shape[1]

    @pl.kernel(
        out_type=jax.ShapeDtypeStruct((m, v), table.dtype),
        mesh=plsc.VectorSubcoreMesh(core_axis_name="core",
                                    subcore_axis_name="subcore"),
    )
    def k(x_hbm, i_hbm, o_hbm):
        def body(i_vmem, o_vmem):
            pltpu.sync_copy(x_hbm.at[i_vmem.at[0]], o_vmem)

        pltpu.emit_pipeline(
            body,
            grid=(m // _GW,),
            in_specs=[pl.BlockSpec((1, _GW), lambda i: (0, i))],
            out_specs=[pl.BlockSpec((_GW, v), lambda i: (i, 0))],
            core_axis_name=("core", "subcore"),
            dimension_semantics=(pltpu.PARALLEL,),
        )(i_hbm, o_hbm)

    return k(table, idx_pad)


def _edge_feat_body(d2_ref, xc_ref, xcj_ref, we_ref, be_ref, eh_ref):
    d2b = d2_ref[...]
    dd = jnp.sqrt(d2b + 1e-8)
    xci = xc_ref[:, 0:3]
    xci = jnp.broadcast_to(xci[:, None, :], (RB, K, 3)).reshape(EB, 3)
    xcj = xcj_ref[...]
    cen = jax.lax.broadcasted_iota(
        jnp.int32, (1, NUM_RBF), 1).astype(jnp.float32) * (20.0 / 15.0)
    sigma = 20.0 / NUM_RBF
    rbf = jnp.exp2(-(((dd - cen) / sigma) ** 2) * _LOG2E)
    dirv = (xcj[:, 0:3] - xci) / (dd + 1e-8)
    ones = jnp.ones((EB, 1), jnp.float32)
    feat = jnp.concatenate([rbf, dirv, ones], axis=1)
    eh_ref[...] = (jnp.dot(feat, we_ref[...],
                           preferred_element_type=jnp.float32) + be_ref[...])


def _edge_feat(d2flat, table0, xcj, we, be, off):
    return pl.pallas_call(
        _edge_feat_body,
        grid=(HBLK,),
        in_specs=[
            pl.BlockSpec((EB, 1), lambda i: (i, 0)),
            pl.BlockSpec((RB, DH), lambda i: (i + off * HBLK, 1)),
            pl.BlockSpec((EB, DH), lambda i: (i, 1)),
            pl.BlockSpec((NUM_RBF + 4, DH), lambda i: (0, 0)),
            pl.BlockSpec((1, DH), lambda i: (0, 0)),
        ],
        out_specs=pl.BlockSpec((EB, DH), lambda i: (i, 0)),
        out_shape=jax.ShapeDtypeStruct((HE, DH), jnp.float32),
    )(d2flat, table0, xcj, we, be)


def _msg_node_body(nh_ref, hj_ref, eh_ref, wm1_ref, bm1_ref, wm2_ref, bm2_ref,
                   wd1_ref, bd1_ref, wd2_ref, bd2_ref, buf_ref, out_ref):
    del buf_ref
    hi = nh_ref[...]
    w1a = wm1_ref[0:DH, :]
    w1b = wm1_ref[DH:2 * DH, :]
    w1c = wm1_ref[2 * DH:3 * DH, :]
    t1 = jnp.dot(hi, w1a, preferred_element_type=jnp.float32) + bm1_ref[...]
    t1r = jnp.broadcast_to(t1[:, None, :], (RB, K, DH)).reshape(EB, DH)
    z = (t1r
         + jnp.dot(hj_ref[...], w1b, preferred_element_type=jnp.float32)
         + jnp.dot(eh_ref[...], w1c, preferred_element_type=jnp.float32))
    m = (jnp.dot(_softplus(z), wm2_ref[...],
                 preferred_element_type=jnp.float32) + bm2_ref[...])
    agg = jnp.sum(m.reshape(RB, K, DH), axis=1) / float(K)
    h = _ln(hi + agg)
    dh = (jnp.dot(_softplus(jnp.dot(h, wd1_ref[...],
                                    preferred_element_type=jnp.float32)
                            + bd1_ref[...]),
                  wd2_ref[...], preferred_element_type=jnp.float32)
          + bd2_ref[...])
    out_ref[...] = _ln(h + dh)


def _msg_node(nh, hj, eh, wm1, bm1, wm2, bm2, wd1, bd1, wd2, bd2, buf, off):
    return pl.pallas_call(
        _msg_node_body,
        grid=(HBLK,),
        in_specs=[
            pl.BlockSpec((RB, DH), lambda i, o=off: (i + o, 0)),
            pl.BlockSpec((EB, DH), lambda i: (i, 0)),
            pl.BlockSpec((EB, DH), lambda i: (i, 0)),
            pl.BlockSpec((3 * DH, DH), lambda i: (0, 0)),
            pl.BlockSpec((1, DH), lambda i: (0, 0)),
            pl.BlockSpec((DH, DH), lambda i: (0, 0)),
            pl.BlockSpec((1, DH), lambda i: (0, 0)),
            pl.BlockSpec((DH, DH), lambda i: (0, 0)),
            pl.BlockSpec((1, DH), lambda i: (0, 0)),
            pl.BlockSpec((DH, DH), lambda i: (0, 0)),
            pl.BlockSpec((1, DH), lambda i: (0, 0)),
            pl.BlockSpec((RB, DH), lambda i, o=off: (i + o, 0)),
        ],
        out_specs=pl.BlockSpec((RB, DH), lambda i, o=off: (i + o, 0)),
        out_shape=jax.ShapeDtypeStruct((N, DH), jnp.float32),
        input_output_aliases={11: 0},
    )(nh, hj, eh, wm1, bm1, wm2, bm2, wd1, bd1, wd2, bd2, buf)


def _edge_upd_body(nh_ref, hj_ref, eh_ref, we1_ref, be1_ref, we2_ref, be2_ref,
                   out_ref):
    hi = nh_ref[...]
    w1a = we1_ref[0:DH, :]
    w1b = we1_ref[DH:2 * DH, :]
    w1c = we1_ref[2 * DH:3 * DH, :]
    t1 = jnp.dot(hi, w1a, preferred_element_type=jnp.float32) + be1_ref[...]
    t1r = jnp.broadcast_to(t1[:, None, :], (RB, K, DH)).reshape(EB, DH)
    eh = eh_ref[...]
    z = (t1r
         + jnp.dot(hj_ref[...], w1b, preferred_element_type=jnp.float32)
         + jnp.dot(eh, w1c, preferred_element_type=jnp.float32))
    de = (jnp.dot(_softplus(z), we2_ref[...],
                  preferred_element_type=jnp.float32) + be2_ref[...])
    out_ref[...] = _ln(eh + de)


class _Reshape3D:
    def __init__(self, ref):
        self._ref = ref

    def __setitem__(self, idx, val):
        self._ref[...] = val.reshape(RB, K, DH)


def _edge_upd_final_body(nh_ref, hj_ref, eh_ref, we1_ref, be1_ref, we2_ref,
                         be2_ref, buf_ref, out_ref):
    del buf_ref
    _edge_upd_body(nh_ref, hj_ref, eh_ref, we1_ref, be1_ref, we2_ref,
                   be2_ref, _Reshape3D(out_ref))


def _edge_upd_final(nh, hj, eh, we1, be1, we2, be2, buf, off):
    return pl.pallas_call(
        _edge_upd_final_body,
        grid=(HBLK,),
        in_specs=[
            pl.BlockSpec((RB, DH), lambda i, o=off: (i + o, 0)),
            pl.BlockSpec((EB, DH), lambda i: (i, 0)),
            pl.BlockSpec((EB, DH), lambda i: (i, 0)),
            pl.BlockSpec((3 * DH, DH), lambda i: (0, 0)),
            pl.BlockSpec((1, DH), lambda i: (0, 0)),
            pl.BlockSpec((DH, DH), lambda i: (0, 0)),
            pl.BlockSpec((1, DH), lambda i: (0, 0)),
            pl.BlockSpec((RB, K, DH), lambda i, o=off: (i + o, 0, 0)),
        ],
        out_specs=pl.BlockSpec((RB, K, DH), lambda i, o=off: (i + o, 0, 0)),
        out_shape=jax.ShapeDtypeStruct((N, K, DH), jnp.float32),
        input_output_aliases={7: 0},
    )(nh, hj, eh, we1, be1, we2, be2, buf)


def _edge_upd(nh, hj, eh, we1, be1, we2, be2, off):
    return pl.pallas_call(
        _edge_upd_body,
        grid=(HBLK,),
        in_specs=[
            pl.BlockSpec((RB, DH), lambda i, o=off: (i + o, 0)),
            pl.BlockSpec((EB, DH), lambda i: (i, 0)),
            pl.BlockSpec((EB, DH), lambda i: (i, 0)),
            pl.BlockSpec((3 * DH, DH), lambda i: (0, 0)),
            pl.BlockSpec((1, DH), lambda i: (0, 0)),
            pl.BlockSpec((DH, DH), lambda i: (0, 0)),
            pl.BlockSpec((1, DH), lambda i: (0, 0)),
        ],
        out_specs=pl.BlockSpec((EB, DH), lambda i: (i, 0)),
        out_shape=jax.ShapeDtypeStruct((HE, DH), jnp.float32),
    )(nh, hj, eh, we1, be1, we2, be2)


def _zeros(shape):
    return jnp.zeros(shape, jnp.float32)


def kernel(X, C, W_node, b_node, W_edge, b_edge, Wm1, bm1, Wm2, bm2,
           Wd1, bd1, Wd2, bd2, We1, be1, We2, be2):
    B = X.shape[0]
    x2 = X.reshape(N, 12)
    x2t_pad = jnp.concatenate(
        [x2.T, jnp.full((12, NP - N), 1e6, jnp.float32)], axis=1)
    x2tc = x2t_pad.reshape(12, NC, 128).transpose(1, 0, 2)
    table0, d2sel, eidx = _knn_embed(
        x2, x2tc, W_node, b_node.reshape(1, DH))

    def pad_idx(half):
        return jnp.concatenate(
            [half.reshape(1, HE), jnp.zeros((1, HEPAD - HE), jnp.int32)],
            axis=1)

    idxA = pad_idx(eidx[:HN])
    idxB = pad_idx(eidx[HN:])

    gA = _gather_rows(table0, idxA)
    gB = _gather_rows(table0, idxB)
    be_ = b_edge.reshape(1, DH)
    ehA = _edge_feat(d2sel[:HN].reshape(HE, 1), table0, gA, W_edge, be_, 0)
    ehB = _edge_feat(d2sel[HN:].reshape(HE, 1), table0, gB, W_edge, be_, 1)

    nh_src = table0
    hjA, hjB = gA, gB
    for l in range(3):
        wl = (Wm1[l], bm1[l].reshape(1, DH), Wm2[l], bm2[l].reshape(1, DH),
              Wd1[l], bd1[l].reshape(1, DH), Wd2[l], bd2[l].reshape(1, DH))
        nh_half = _msg_node(nh_src, hjA, ehA, *wl, _zeros((N, DH)), 0)
        nh_full = _msg_node(nh_src, hjB, ehB, *wl, nh_half, HBLK)
        hjA = _gather_rows(nh_full, idxA)
        hjB = _gather_rows(nh_full, idxB)
        el = (We1[l], be1[l].reshape(1, DH), We2[l], be2[l].reshape(1, DH))
        if l < 2:
            ehA = _edge_upd(nh_full, hjA, ehA, *el, 0)
            ehB = _edge_upd(nh_full, hjB, ehB, *el, HBLK)
        else:
            eh = _edge_upd_final(nh_full, hjA, ehA, *el,
                                 _zeros((N, K, DH)), 0)
            eh = _edge_upd_final(nh_full, hjB, ehB, *el, eh, HBLK)
        nh_src = nh_full

    mask_i = (C > 0).astype(jnp.float32)
    mask_ij = jnp.broadcast_to(mask_i[:, :, None], (B, N, K))
    return (nh_full.reshape(B, N, DH),
            eh.reshape(B, N, K, DH),
            eidx.reshape(B, N, K),
            mask_i,
            mask_ij)


# speedup vs baseline: 1.0700x; 1.0513x over previous
import functools

import jax
import jax.numpy as jnp
from jax.experimental import pallas as pl
from jax.experimental.pallas import tpu as pltpu
from jax.experimental.pallas import tpu_sc as plsc

N = 10000
K = 30
DH = 128
NUM_RBF = 16
E = N * K
RB = 200
EB = RB * K
NBLK = N // RB
HN = N // 2
HE = HN * K
HBLK = NBLK // 2
HEPAD = 151552


_LOG2E = 1.4426950408889634
_LN2 = 0.6931471805599453


def _softplus(x):
    return jnp.maximum(x, 0.0) + _LN2 * jnp.log2(
        1.0 + jnp.exp2(-jnp.abs(x) * _LOG2E))


def _ln(x):
    mu = jnp.mean(x, axis=-1, keepdims=True)
    var = jnp.mean((x - mu) ** 2, axis=-1, keepdims=True)
    return (x - mu) / jnp.sqrt(var + 1e-5)


NP = 10240
NC = NP // 128
DEPTH = 8


def _knn_embed_body(x2_ref, x2tc_ref, wn_ref, bn_ref,
                    tab_ref, d2sel_ref, eidx_ref, vstk_ref, cstk_ref):
    i = pl.program_id(0)
    x2 = x2_ref[...]

    xr = (x2[:, 0:1] + x2[:, 3:4] + x2[:, 6:7] + x2[:, 9:10]) * 0.25
    yr = (x2[:, 1:2] + x2[:, 4:5] + x2[:, 7:8] + x2[:, 10:11]) * 0.25
    zr = (x2[:, 2:3] + x2[:, 5:6] + x2[:, 8:9] + x2[:, 11:12]) * 0.25

    row_ids = i * RB + jax.lax.broadcasted_iota(jnp.int32, (RB, 1), 0)
    lane = jax.lax.broadcasted_iota(jnp.int32, (1, 128), 1)

    for t in range(DEPTH):
        vstk_ref[t] = jnp.full((RB, 128), jnp.inf, jnp.float32)
        cstk_ref[t] = jnp.full((RB, 128), 2 ** 30, jnp.int32)

    def chunk_body(s, _):
        for half in range(2):
            c = 2 * s + half
            xt2 = x2tc_ref[c]
            xT = (xt2[0:1] + xt2[3:4] + xt2[6:7] + xt2[9:10]) * 0.25
            yT = (xt2[1:2] + xt2[4:5] + xt2[7:8] + xt2[10:11]) * 0.25
            zT = (xt2[2:3] + xt2[5:6] + xt2[8:9] + xt2[11:12]) * 0.25
            dx = xr - xT
            dy = yr - yT
            dz = zr - zT
            d2c = dx * dx + dy * dy + dz * dz
            colnum = c * 128 + lane
            d2c = jnp.where(row_ids == colnum, 1e9, d2c)
            v = d2c
            cc = jnp.broadcast_to(colnum, (RB, 128))
            for t in range(DEPTH):
                vt = vstk_ref[t]
                ct = cstk_ref[t]
                swap = v < vt
                vstk_ref[t] = jnp.where(swap, v, vt)
                cstk_ref[t] = jnp.where(swap, cc, ct)
                v = jnp.where(swap, vt, v)
                cc = jnp.where(swap, ct, cc)
        return 0

    jax.lax.fori_loop(0, NC // 2, chunk_body, 0)

    lane_k = jax.lax.broadcasted_iota(jnp.int32, (1, K), 1)

    def body(k, carry):
        vals, idxs = carry
        vall = vstk_ref[...]
        call = cstk_ref[...]
        m2 = jnp.min(vall, axis=0)
        m = jnp.min(m2, axis=1, keepdims=True)
        cand = jnp.where(vall == m[None], call, jnp.int32(2 ** 30))
        s2 = jnp.min(cand, axis=0)
        sel = jnp.min(s2, axis=1, keepdims=True)
        vstk_ref[...] = jnp.where(call == sel[None], jnp.inf, vall)
        vals = jnp.where(lane_k == k, m, vals)
        idxs = jnp.where(lane_k == k, sel, idxs)
        return vals, idxs

    vals0 = jnp.zeros((RB, K), jnp.float32)
    idxs0 = jnp.zeros((RB, K), jnp.int32)
    vals, idxs = jax.lax.fori_loop(0, K, body, (vals0, idxs0))
    d2sel_ref[...] = vals
    eidx_ref[...] = idxs

    xc3 = jnp.concatenate([xr, yr, zr], axis=1)
    xrel = x2 - jnp.concatenate([xc3, xc3, xc3, xc3], axis=1)
    bond = x2[:, 3:12] - x2[:, 0:9]
    lls = []
    for t in range(3):
        b = bond[:, 3 * t:3 * t + 3]
        sq = jnp.sum(b * b, axis=1, keepdims=True)
        lls.append(jnp.log(jnp.sqrt(sq + 1e-8) + 1e-6))
    feat = jnp.concatenate([xrel] + lls, axis=1)
    nh = (jnp.dot(feat, wn_ref[...],
                  preferred_element_type=jnp.float32) + bn_ref[...])
    tab_ref[...] = jnp.concatenate(
        [nh, xc3, jnp.zeros((RB, 125), jnp.float32)], axis=1)


def _knn_embed(x2, x2tc, wn, bn):
    return pl.pallas_call(
        _knn_embed_body,
        grid=(NBLK,),
        in_specs=[
            pl.BlockSpec((RB, 12), lambda i: (i, 0)),
            pl.BlockSpec((NC, 12, 128), lambda i: (0, 0, 0)),
            pl.BlockSpec((15, DH), lambda i: (0, 0)),
            pl.BlockSpec((1, DH), lambda i: (0, 0)),
        ],
        out_specs=[
            pl.BlockSpec((RB, 2 * DH), lambda i: (i, 0)),
            pl.BlockSpec((RB, K), lambda i: (i, 0)),
            pl.BlockSpec((RB, K), lambda i: (i, 0)),
        ],
        out_shape=[
            jax.ShapeDtypeStruct((N, 2 * DH), jnp.float32),
            jax.ShapeDtypeStruct((N, K), jnp.float32),
            jax.ShapeDtypeStruct((N, K), jnp.int32),
        ],
        scratch_shapes=[pltpu.VMEM((DEPTH, RB, 128), jnp.float32),
                        pltpu.VMEM((DEPTH, RB, 128), jnp.int32)],
    )(x2, x2tc, wn, bn)


_GW = 128


def _gather_rows(table, idx_pad):
    v = table.shape[1]
    m = idx_pad.shape[1]

    @pl.kernel(
        out_type=jax.ShapeDtypeStruct((m, v), table.dtype),
        mesh=plsc.VectorSubcoreMesh(core_axis_name="core",
                                    subcore_axis_name="subcore"),
    )
    def k(x_hbm, i_hbm, o_hbm):
        def body(i_vmem, o_vmem):
            pltpu.sync_copy(x_hbm.at[i_vmem.at[0]], o_vmem)

        pltpu.emit_pipeline(
            body,
            grid=(m // _GW,),
            in_specs=[pl.BlockSpec((1, _GW), lambda i: (0, i))],
            out_specs=[pl.BlockSpec((_GW, v), lambda i: (i, 0))],
            core_axis_name=("core", "subcore"),
            dimension_semantics=(pltpu.PARALLEL,),
        )(i_hbm, o_hbm)

    return k(table, idx_pad)


def _edge_feat_body(d2_ref, xc_ref, xcj_ref, we_ref, be_ref, eh_ref):
    d2b = d2_ref[...]
    dd = jnp.sqrt(d2b + 1e-8)
    xci = xc_ref[:, 0:3]
    xci = jnp.broadcast_to(xci[:, None, :], (RB, K, 3)).reshape(EB, 3)
    xcj = xcj_ref[...]
    cen = jax.lax.broadcasted_iota(
        jnp.int32, (1, NUM_RBF), 1).astype(jnp.float32) * (20.0 / 15.0)
    sigma = 20.0 / NUM_RBF
    rbf = jnp.exp2(-(((dd - cen) / sigma) ** 2) * _LOG2E)
    dirv = (xcj[:, 0:3] - xci) / (dd + 1e-8)
    ones = jnp.ones((EB, 1), jnp.float32)
    feat = jnp.concatenate([rbf, dirv, ones], axis=1)
    eh_ref[...] = (jnp.dot(feat, we_ref[...],
                           preferred_element_type=jnp.float32) + be_ref[...])


def _edge_feat(d2flat, table0, xcj, we, be, off):
    return pl.pallas_call(
        _edge_feat_body,
        grid=(HBLK,),
        in_specs=[
            pl.BlockSpec((EB, 1), lambda i: (i, 0)),
            pl.BlockSpec((RB, DH), lambda i: (i + off * HBLK, 1)),
            pl.BlockSpec((EB, DH), lambda i: (i, 1)),
            pl.BlockSpec((NUM_RBF + 4, DH), lambda i: (0, 0)),
            pl.BlockSpec((1, DH), lambda i: (0, 0)),
        ],
        out_specs=pl.BlockSpec((EB, DH), lambda i: (i, 0)),
        out_shape=jax.ShapeDtypeStruct((HE, DH), jnp.float32),
    )(d2flat, table0, xcj, we, be)


def _msg_node_body(nh_ref, hj_ref, eh_ref, wm1_ref, bm1_ref, wm2_ref, bm2_ref,
                   wd1_ref, bd1_ref, wd2_ref, bd2_ref, buf_ref, out_ref):
    del buf_ref
    hi = nh_ref[...]
    w1a = wm1_ref[0:DH, :]
    w1b = wm1_ref[DH:2 * DH, :]
    w1c = wm1_ref[2 * DH:3 * DH, :]
    t1 = jnp.dot(hi, w1a, preferred_element_type=jnp.float32) + bm1_ref[...]
    t1r = jnp.broadcast_to(t1[:, None, :], (RB, K, DH)).reshape(EB, DH)
    z = (t1r
         + jnp.dot(hj_ref[...], w1b, preferred_element_type=jnp.float32)
         + jnp.dot(eh_ref[...], w1c, preferred_element_type=jnp.float32))
    m = (jnp.dot(_softplus(z), wm2_ref[...],
                 preferred_element_type=jnp.float32) + bm2_ref[...])
    agg = jnp.sum(m.reshape(RB, K, DH), axis=1) / float(K)
    h = _ln(hi + agg)
    dh = (jnp.dot(_softplus(jnp.dot(h, wd1_ref[...],
                                    preferred_element_type=jnp.float32)
                            + bd1_ref[...]),
                  wd2_ref[...], preferred_element_type=jnp.float32)
          + bd2_ref[...])
    out_ref[...] = _ln(h + dh)


def _msg_node(nh, hj, eh, wm1, bm1, wm2, bm2, wd1, bd1, wd2, bd2, buf, off):
    return pl.pallas_call(
        _msg_node_body,
        grid=(HBLK,),
        in_specs=[
            pl.BlockSpec((RB, DH), lambda i, o=off: (i + o, 0)),
            pl.BlockSpec((EB, DH), lambda i: (i, 0)),
            pl.BlockSpec((EB, DH), lambda i: (i, 0)),
            pl.BlockSpec((3 * DH, DH), lambda i: (0, 0)),
            pl.BlockSpec((1, DH), lambda i: (0, 0)),
            pl.BlockSpec((DH, DH), lambda i: (0, 0)),
            pl.BlockSpec((1, DH), lambda i: (0, 0)),
            pl.BlockSpec((DH, DH), lambda i: (0, 0)),
            pl.BlockSpec((1, DH), lambda i: (0, 0)),
            pl.BlockSpec((DH, DH), lambda i: (0, 0)),
            pl.BlockSpec((1, DH), lambda i: (0, 0)),
            pl.BlockSpec((RB, DH), lambda i, o=off: (i + o, 0)),
        ],
        out_specs=pl.BlockSpec((RB, DH), lambda i, o=off: (i + o, 0)),
        out_shape=jax.ShapeDtypeStruct((N, DH), jnp.float32),
        input_output_aliases={11: 0},
    )(nh, hj, eh, wm1, bm1, wm2, bm2, wd1, bd1, wd2, bd2, buf)


def _edge_upd_body(nh_ref, hj_ref, eh_ref, we1_ref, be1_ref, we2_ref, be2_ref,
                   out_ref):
    hi = nh_ref[...]
    w1a = we1_ref[0:DH, :]
    w1b = we1_ref[DH:2 * DH, :]
    w1c = we1_ref[2 * DH:3 * DH, :]
    t1 = jnp.dot(hi, w1a, preferred_element_type=jnp.float32) + be1_ref[...]
    t1r = jnp.broadcast_to(t1[:, None, :], (RB, K, DH)).reshape(EB, DH)
    eh = eh_ref[...]
    z = (t1r
         + jnp.dot(hj_ref[...], w1b, preferred_element_type=jnp.float32)
         + jnp.dot(eh, w1c, preferred_element_type=jnp.float32))
    de = (jnp.dot(_softplus(z), we2_ref[...],
                  preferred_element_type=jnp.float32) + be2_ref[...])
    out_ref[...] = _ln(eh + de)


class _Reshape3D:
    def __init__(self, ref):
        self._ref = ref

    def __setitem__(self, idx, val):
        self._ref[...] = val.reshape(RB, K, DH)


def _edge_upd_final_body(nh_ref, hj_ref, eh_ref, we1_ref, be1_ref, we2_ref,
                         be2_ref, buf_ref, out_ref):
    del buf_ref
    _edge_upd_body(nh_ref, hj_ref, eh_ref, we1_ref, be1_ref, we2_ref,
                   be2_ref, _Reshape3D(out_ref))


def _edge_upd_final(nh, hj, eh, we1, be1, we2, be2, buf, off):
    return pl.pallas_call(
        _edge_upd_final_body,
        grid=(HBLK,),
        in_specs=[
            pl.BlockSpec((RB, DH), lambda i, o=off: (i + o, 0)),
            pl.BlockSpec((EB, DH), lambda i: (i, 0)),
            pl.BlockSpec((EB, DH), lambda i: (i, 0)),
            pl.BlockSpec((3 * DH, DH), lambda i: (0, 0)),
            pl.BlockSpec((1, DH), lambda i: (0, 0)),
            pl.BlockSpec((DH, DH), lambda i: (0, 0)),
            pl.BlockSpec((1, DH), lambda i: (0, 0)),
            pl.BlockSpec((RB, K, DH), lambda i, o=off: (i + o, 0, 0)),
        ],
        out_specs=pl.BlockSpec((RB, K, DH), lambda i, o=off: (i + o, 0, 0)),
        out_shape=jax.ShapeDtypeStruct((N, K, DH), jnp.float32),
        input_output_aliases={7: 0},
    )(nh, hj, eh, we1, be1, we2, be2, buf)


def _edge_upd(nh, hj, eh, we1, be1, we2, be2, off):
    return pl.pallas_call(
        _edge_upd_body,
        grid=(HBLK,),
        in_specs=[
            pl.BlockSpec((RB, DH), lambda i, o=off: (i + o, 0)),
            pl.BlockSpec((EB, DH), lambda i: (i, 0)),
            pl.BlockSpec((EB, DH), lambda i: (i, 0)),
            pl.BlockSpec((3 * DH, DH), lambda i: (0, 0)),
            pl.BlockSpec((1, DH), lambda i: (0, 0)),
            pl.BlockSpec((DH, DH), lambda i: (0, 0)),
            pl.BlockSpec((1, DH), lambda i: (0, 0)),
        ],
        out_specs=pl.BlockSpec((EB, DH), lambda i: (i, 0)),
        out_shape=jax.ShapeDtypeStruct((HE, DH), jnp.float32),
    )(nh, hj, eh, we1, be1, we2, be2)


def _zeros(shape):
    return jnp.zeros(shape, jnp.float32)


def kernel(X, C, W_node, b_node, W_edge, b_edge, Wm1, bm1, Wm2, bm2,
           Wd1, bd1, Wd2, bd2, We1, be1, We2, be2):
    B = X.shape[0]
    x2 = X.reshape(N, 12)
    x2t_pad = jnp.concatenate(
        [x2.T, jnp.full((12, NP - N), 1e6, jnp.float32)], axis=1)
    x2tc = x2t_pad.reshape(12, NC, 128).transpose(1, 0, 2)
    table0, d2sel, eidx = _knn_embed(
        x2, x2tc, W_node, b_node.reshape(1, DH))

    def pad_idx(half):
        return jnp.concatenate(
            [half.reshape(1, HE), jnp.zeros((1, HEPAD - HE), jnp.int32)],
            axis=1)

    idxA = pad_idx(eidx[:HN])
    idxB = pad_idx(eidx[HN:])

    gA = _gather_rows(table0, idxA)
    gB = _gather_rows(table0, idxB)
    be_ = b_edge.reshape(1, DH)
    ehA = _edge_feat(d2sel[:HN].reshape(HE, 1), table0, gA, W_edge, be_, 0)
    ehB = _edge_feat(d2sel[HN:].reshape(HE, 1), table0, gB, W_edge, be_, 1)

    nh_src = table0
    hjA, hjB = gA, gB
    for l in range(3):
        wl = (Wm1[l], bm1[l].reshape(1, DH), Wm2[l], bm2[l].reshape(1, DH),
              Wd1[l], bd1[l].reshape(1, DH), Wd2[l], bd2[l].reshape(1, DH))
        nh_half = _msg_node(nh_src, hjA, ehA, *wl, _zeros((N, DH)), 0)
        nh_full = _msg_node(nh_src, hjB, ehB, *wl, nh_half, HBLK)
        hjA = _gather_rows(nh_full, idxA)
        hjB = _gather_rows(nh_full, idxB)
        el = (We1[l], be1[l].reshape(1, DH), We2[l], be2[l].reshape(1, DH))
        if l < 2:
            ehA = _edge_upd(nh_full, hjA, ehA, *el, 0)
            ehB = _edge_upd(nh_full, hjB, ehB, *el, HBLK)
        else:
            eh = _edge_upd_final(nh_full, hjA, ehA, *el,
                                 _zeros((N, K, DH)), 0)
            eh = _edge_upd_final(nh_full, hjB, ehB, *el, eh, HBLK)
        nh_src = nh_full

    mask_i = (C > 0).astype(jnp.float32)
    mask_ij = jnp.broadcast_to(mask_i[:, :, None], (B, N, K))
    return (nh_full.reshape(B, N, DH),
            eh.reshape(B, N, K, DH),
            eidx.reshape(B, N, K),
            mask_i,
            mask_ij)


# speedup vs baseline: 1.0705x; 1.0004x over previous
import jax
import jax.numpy as jnp
from jax.experimental import pallas as pl
from jax.experimental.pallas import tpu as pltpu
from jax.experimental.pallas import tpu_sc as plsc

N = 10000
K = 30
DH = 128
NUM_RBF = 16
E = N * K
RB = 200
EB = RB * K
NBLK = N // RB
HN = N // 2
HE = HN * K
HBLK = NBLK // 2
HEPAD = 151552


_LOG2E = 1.4426950408889634
_LN2 = 0.6931471805599453


def _softplus(x):
    return jnp.maximum(x, 0.0) + _LN2 * jnp.log2(
        1.0 + jnp.exp2(-jnp.abs(x) * _LOG2E))


def _ln(x):
    mu = jnp.mean(x, axis=-1, keepdims=True)
    var = jnp.mean((x - mu) ** 2, axis=-1, keepdims=True)
    return (x - mu) / jnp.sqrt(var + 1e-5)


NP = 10240
NC = NP // 128
DEPTH = 8


def _knn_embed_body(x2_ref, x2tc_ref, wn_ref, bn_ref,
                    tab_ref, d2sel_ref, eidx_ref, vstk_ref, cstk_ref):
    i = pl.program_id(0)
    x2 = x2_ref[...]

    xr = (x2[:, 0:1] + x2[:, 3:4] + x2[:, 6:7] + x2[:, 9:10]) * 0.25
    yr = (x2[:, 1:2] + x2[:, 4:5] + x2[:, 7:8] + x2[:, 10:11]) * 0.25
    zr = (x2[:, 2:3] + x2[:, 5:6] + x2[:, 8:9] + x2[:, 11:12]) * 0.25

    row_ids = i * RB + jax.lax.broadcasted_iota(jnp.int32, (RB, 1), 0)
    lane = jax.lax.broadcasted_iota(jnp.int32, (1, 128), 1)

    for t in range(DEPTH):
        vstk_ref[t] = jnp.full((RB, 128), jnp.inf, jnp.float32)
        cstk_ref[t] = jnp.full((RB, 128), 2 ** 30, jnp.int32)

    def chunk_body(s, _):
        for half in range(2):
            c = 2 * s + half
            xt2 = x2tc_ref[c]
            xT = (xt2[0:1] + xt2[3:4] + xt2[6:7] + xt2[9:10]) * 0.25
            yT = (xt2[1:2] + xt2[4:5] + xt2[7:8] + xt2[10:11]) * 0.25
            zT = (xt2[2:3] + xt2[5:6] + xt2[8:9] + xt2[11:12]) * 0.25
            dx = xr - xT
            dy = yr - yT
            dz = zr - zT
            d2c = dx * dx + dy * dy + dz * dz
            colnum = c * 128 + lane
            d2c = jnp.where(row_ids == colnum, 1e9, d2c)
            v = d2c
            cc = jnp.broadcast_to(colnum, (RB, 128))
            for t in range(DEPTH):
                vt = vstk_ref[t]
                ct = cstk_ref[t]
                swap = v < vt
                vstk_ref[t] = jnp.where(swap, v, vt)
                cstk_ref[t] = jnp.where(swap, cc, ct)
                v = jnp.where(swap, vt, v)
                cc = jnp.where(swap, ct, cc)
        return 0

    jax.lax.fori_loop(0, NC // 2, chunk_body, 0)

    lane_k = jax.lax.broadcasted_iota(jnp.int32, (1, K), 1)

    def body(k, carry):
        vals, idxs = carry
        vall = vstk_ref[...]
        call = cstk_ref[...]
        m2 = jnp.min(vall, axis=0)
        m = jnp.min(m2, axis=1, keepdims=True)
        cand = jnp.where(vall == m[None], call, jnp.int32(2 ** 30))
        s2 = jnp.min(cand, axis=0)
        sel = jnp.min(s2, axis=1, keepdims=True)
        vstk_ref[...] = jnp.where(call == sel[None], jnp.inf, vall)
        vals = jnp.where(lane_k == k, m, vals)
        idxs = jnp.where(lane_k == k, sel, idxs)
        return vals, idxs

    vals0 = jnp.zeros((RB, K), jnp.float32)
    idxs0 = jnp.zeros((RB, K), jnp.int32)
    vals, idxs = jax.lax.fori_loop(0, K, body, (vals0, idxs0))
    d2sel_ref[...] = vals
    eidx_ref[...] = idxs

    xc3 = jnp.concatenate([xr, yr, zr], axis=1)
    xrel = x2 - jnp.concatenate([xc3, xc3, xc3, xc3], axis=1)
    bond = x2[:, 3:12] - x2[:, 0:9]
    lls = []
    for t in range(3):
        b = bond[:, 3 * t:3 * t + 3]
        sq = jnp.sum(b * b, axis=1, keepdims=True)
        lls.append(jnp.log(jnp.sqrt(sq + 1e-8) + 1e-6))
    feat = jnp.concatenate([xrel] + lls, axis=1)
    nh = (jnp.dot(feat, wn_ref[...],
                  preferred_element_type=jnp.float32) + bn_ref[...])
    tab_ref[...] = jnp.concatenate(
        [nh, xc3, jnp.zeros((RB, 125), jnp.float32)], axis=1)


def _knn_embed(x2, x2tc, wn, bn):
    return pl.pallas_call(
        _knn_embed_body,
        grid=(NBLK,),
        in_specs=[
            pl.BlockSpec((RB, 12), lambda i: (i, 0)),
            pl.BlockSpec((NC, 12, 128), lambda i: (0, 0, 0)),
            pl.BlockSpec((15, DH), lambda i: (0, 0)),
            pl.BlockSpec((1, DH), lambda i: (0, 0)),
        ],
        out_specs=[
            pl.BlockSpec((RB, 2 * DH), lambda i: (i, 0)),
            pl.BlockSpec((RB, K), lambda i: (i, 0)),
            pl.BlockSpec((RB, K), lambda i: (i, 0)),
        ],
        out_shape=[
            jax.ShapeDtypeStruct((N, 2 * DH), jnp.float32),
            jax.ShapeDtypeStruct((N, K), jnp.float32),
            jax.ShapeDtypeStruct((N, K), jnp.int32),
        ],
        scratch_shapes=[pltpu.VMEM((DEPTH, RB, 128), jnp.float32),
                        pltpu.VMEM((DEPTH, RB, 128), jnp.int32)],
    )(x2, x2tc, wn, bn)


_GW = 128


def _gather_rows(table, idx_pad):
    v = table.shape[1]
    m = idx_pad.shape[1]

    @pl.kernel(
        out_type=jax.ShapeDtypeStruct((m, v), table.dtype),
        mesh=plsc.VectorSubcoreMesh(core_axis_name="core",
                                    subcore_axis_name="subcore"),
    )
    def k(x_hbm, i_hbm, o_hbm):
        def body(i_vmem, o_vmem):
            pltpu.sync_copy(x_hbm.at[i_vmem.at[0]], o_vmem)

        pltpu.emit_pipeline(
            body,
            grid=(m // _GW,),
            in_specs=[pl.BlockSpec((1, _GW), lambda i: (0, i))],
            out_specs=[pl.BlockSpec((_GW, v), lambda i: (i, 0))],
            core_axis_name=("core", "subcore"),
            dimension_semantics=(pltpu.PARALLEL,),
        )(i_hbm, o_hbm)

    return k(table, idx_pad)


def _edge_feat_body(d2_ref, xc_ref, xcj_ref, we_ref, be_ref, eh_ref):
    d2b = d2_ref[...]
    dd = jnp.sqrt(d2b + 1e-8)
    xci = xc_ref[:, 0:3]
    xci = jnp.broadcast_to(xci[:, None, :], (RB, K, 3)).reshape(EB, 3)
    xcj = xcj_ref[...]
    cen = jax.lax.broadcasted_iota(
        jnp.int32, (1, NUM_RBF), 1).astype(jnp.float32) * (20.0 / 15.0)
    sigma = 20.0 / NUM_RBF
    rbf = jnp.exp2(-(((dd - cen) / sigma) ** 2) * _LOG2E)
    dirv = (xcj[:, 0:3] - xci) / (dd + 1e-8)
    ones = jnp.ones((EB, 1), jnp.float32)
    feat = jnp.concatenate([rbf, dirv, ones], axis=1)
    eh_ref[...] = (jnp.dot(feat, we_ref[...],
                           preferred_element_type=jnp.float32) + be_ref[...])


def _edge_feat(d2flat, table0, xcj, we, be, off):
    return pl.pallas_call(
        _edge_feat_body,
        grid=(HBLK,),
        in_specs=[
            pl.BlockSpec((EB, 1), lambda i: (i, 0)),
            pl.BlockSpec((RB, DH), lambda i: (i + off * HBLK, 1)),
            pl.BlockSpec((EB, DH), lambda i: (i, 1)),
            pl.BlockSpec((NUM_RBF + 4, DH), lambda i: (0, 0)),
            pl.BlockSpec((1, DH), lambda i: (0, 0)),
        ],
        out_specs=pl.BlockSpec((EB, DH), lambda i: (i, 0)),
        out_shape=jax.ShapeDtypeStruct((HE, DH), jnp.float32),
    )(d2flat, table0, xcj, we, be)


def _msg_node_body(nh_ref, hj_ref, eh_ref, wm1_ref, bm1_ref, wm2_ref, bm2_ref,
                   wd1_ref, bd1_ref, wd2_ref, bd2_ref, buf_ref, out_ref):
    del buf_ref
    hi = nh_ref[...]
    w1a = wm1_ref[0:DH, :]
    w1b = wm1_ref[DH:2 * DH, :]
    w1c = wm1_ref[2 * DH:3 * DH, :]
    t1 = jnp.dot(hi, w1a, preferred_element_type=jnp.float32) + bm1_ref[...]
    t1r = jnp.broadcast_to(t1[:, None, :], (RB, K, DH)).reshape(EB, DH)
    z = (t1r
         + jnp.dot(hj_ref[...], w1b, preferred_element_type=jnp.float32)
         + jnp.dot(eh_ref[...], w1c, preferred_element_type=jnp.float32))
    m = (jnp.dot(_softplus(z), wm2_ref[...],
                 preferred_element_type=jnp.float32) + bm2_ref[...])
    agg = jnp.sum(m.reshape(RB, K, DH), axis=1) / float(K)
    h = _ln(hi + agg)
    dh = (jnp.dot(_softplus(jnp.dot(h, wd1_ref[...],
                                    preferred_element_type=jnp.float32)
                            + bd1_ref[...]),
                  wd2_ref[...], preferred_element_type=jnp.float32)
          + bd2_ref[...])
    out_ref[...] = _ln(h + dh)


def _msg_node(nh, hj, eh, wm1, bm1, wm2, bm2, wd1, bd1, wd2, bd2, buf, off):
    return pl.pallas_call(
        _msg_node_body,
        grid=(HBLK,),
        in_specs=[
            pl.BlockSpec((RB, DH), lambda i, o=off: (i + o, 0)),
            pl.BlockSpec((EB, DH), lambda i: (i, 0)),
            pl.BlockSpec((EB, DH), lambda i: (i, 0)),
            pl.BlockSpec((3 * DH, DH), lambda i: (0, 0)),
            pl.BlockSpec((1, DH), lambda i: (0, 0)),
            pl.BlockSpec((DH, DH), lambda i: (0, 0)),
            pl.BlockSpec((1, DH), lambda i: (0, 0)),
            pl.BlockSpec((DH, DH), lambda i: (0, 0)),
            pl.BlockSpec((1, DH), lambda i: (0, 0)),
            pl.BlockSpec((DH, DH), lambda i: (0, 0)),
            pl.BlockSpec((1, DH), lambda i: (0, 0)),
            pl.BlockSpec((RB, DH), lambda i, o=off: (i + o, 0)),
        ],
        out_specs=pl.BlockSpec((RB, DH), lambda i, o=off: (i + o, 0)),
        out_shape=jax.ShapeDtypeStruct((N, DH), jnp.float32),
        input_output_aliases={11: 0},
    )(nh, hj, eh, wm1, bm1, wm2, bm2, wd1, bd1, wd2, bd2, buf)


def _edge_upd_body(nh_ref, hj_ref, eh_ref, we1_ref, be1_ref, we2_ref, be2_ref,
                   out_ref):
    hi = nh_ref[...]
    w1a = we1_ref[0:DH, :]
    w1b = we1_ref[DH:2 * DH, :]
    w1c = we1_ref[2 * DH:3 * DH, :]
    t1 = jnp.dot(hi, w1a, preferred_element_type=jnp.float32) + be1_ref[...]
    t1r = jnp.broadcast_to(t1[:, None, :], (RB, K, DH)).reshape(EB, DH)
    eh = eh_ref[...]
    z = (t1r
         + jnp.dot(hj_ref[...], w1b, preferred_element_type=jnp.float32)
         + jnp.dot(eh, w1c, preferred_element_type=jnp.float32))
    de = (jnp.dot(_softplus(z), we2_ref[...],
                  preferred_element_type=jnp.float32) + be2_ref[...])
    out_ref[...] = _ln(eh + de)


class _Reshape3D:
    def __init__(self, ref):
        self._ref = ref

    def __setitem__(self, idx, val):
        self._ref[...] = val.reshape(RB, K, DH)


def _edge_upd_final_body(nh_ref, hj_ref, eh_ref, we1_ref, be1_ref, we2_ref,
                         be2_ref, buf_ref, out_ref):
    del buf_ref
    _edge_upd_body(nh_ref, hj_ref, eh_ref, we1_ref, be1_ref, we2_ref,
                   be2_ref, _Reshape3D(out_ref))


def _edge_upd_final(nh, hj, eh, we1, be1, we2, be2, buf, off):
    return pl.pallas_call(
        _edge_upd_final_body,
        grid=(HBLK,),
        in_specs=[
            pl.BlockSpec((RB, DH), lambda i, o=off: (i + o, 0)),
            pl.BlockSpec((EB, DH), lambda i: (i, 0)),
            pl.BlockSpec((EB, DH), lambda i: (i, 0)),
            pl.BlockSpec((3 * DH, DH), lambda i: (0, 0)),
            pl.BlockSpec((1, DH), lambda i: (0, 0)),
            pl.BlockSpec((DH, DH), lambda i: (0, 0)),
            pl.BlockSpec((1, DH), lambda i: (0, 0)),
            pl.BlockSpec((RB, K, DH), lambda i, o=off: (i + o, 0, 0)),
        ],
        out_specs=pl.BlockSpec((RB, K, DH), lambda i, o=off: (i + o, 0, 0)),
        out_shape=jax.ShapeDtypeStruct((N, K, DH), jnp.float32),
        input_output_aliases={7: 0},
    )(nh, hj, eh, we1, be1, we2, be2, buf)


def _edge_upd(nh, hj, eh, we1, be1, we2, be2, off):
    return pl.pallas_call(
        _edge_upd_body,
        grid=(HBLK,),
        in_specs=[
            pl.BlockSpec((RB, DH), lambda i, o=off: (i + o, 0)),
            pl.BlockSpec((EB, DH), lambda i: (i, 0)),
            pl.BlockSpec((EB, DH), lambda i: (i, 0)),
            pl.BlockSpec((3 * DH, DH), lambda i: (0, 0)),
            pl.BlockSpec((1, DH), lambda i: (0, 0)),
            pl.BlockSpec((DH, DH), lambda i: (0, 0)),
            pl.BlockSpec((1, DH), lambda i: (0, 0)),
        ],
        out_specs=pl.BlockSpec((EB, DH), lambda i: (i, 0)),
        out_shape=jax.ShapeDtypeStruct((HE, DH), jnp.float32),
    )(nh, hj, eh, we1, be1, we2, be2)


def _zeros(shape):
    return jnp.zeros(shape, jnp.float32)


def kernel(X, C, W_node, b_node, W_edge, b_edge, Wm1, bm1, Wm2, bm2,
           Wd1, bd1, Wd2, bd2, We1, be1, We2, be2):
    B = X.shape[0]
    x2 = X.reshape(N, 12)
    x2t_pad = jnp.concatenate(
        [x2.T, jnp.full((12, NP - N), 1e6, jnp.float32)], axis=1)
    x2tc = x2t_pad.reshape(12, NC, 128).transpose(1, 0, 2)
    table0, d2sel, eidx = _knn_embed(
        x2, x2tc, W_node, b_node.reshape(1, DH))

    def pad_idx(half):
        return jnp.concatenate(
            [half.reshape(1, HE), jnp.zeros((1, HEPAD - HE), jnp.int32)],
            axis=1)

    idxA = pad_idx(eidx[:HN])
    idxB = pad_idx(eidx[HN:])

    gA = _gather_rows(table0, idxA)
    gB = _gather_rows(table0, idxB)
    be_ = b_edge.reshape(1, DH)
    ehA = _edge_feat(d2sel[:HN].reshape(HE, 1), table0, gA, W_edge, be_, 0)
    ehB = _edge_feat(d2sel[HN:].reshape(HE, 1), table0, gB, W_edge, be_, 1)

    nh_src = table0
    hjA, hjB = gA, gB
    for l in range(3):
        wl = (Wm1[l], bm1[l].reshape(1, DH), Wm2[l], bm2[l].reshape(1, DH),
              Wd1[l], bd1[l].reshape(1, DH), Wd2[l], bd2[l].reshape(1, DH))
        nh_half = _msg_node(nh_src, hjA, ehA, *wl, _zeros((N, DH)), 0)
        nh_full = _msg_node(nh_src, hjB, ehB, *wl, nh_half, HBLK)
        hjA = _gather_rows(nh_full, idxA)
        hjB = _gather_rows(nh_full, idxB)
        el = (We1[l], be1[l].reshape(1, DH), We2[l], be2[l].reshape(1, DH))
        if l < 2:
            ehA = _edge_upd(nh_full, hjA, ehA, *el, 0)
            ehB = _edge_upd(nh_full, hjB, ehB, *el, HBLK)
        else:
            eh = _edge_upd_final(nh_full, hjA, ehA, *el,
                                 _zeros((N, K, DH)), 0)
            eh = _edge_upd_final(nh_full, hjB, ehB, *el, eh, HBLK)
        nh_src = nh_full

    mask_i = (C > 0).astype(jnp.float32)
    mask_ij = jnp.broadcast_to(mask_i[:, :, None], (B, N, K))
    return (nh_full.reshape(B, N, DH),
            eh.reshape(B, N, K, DH),
            eidx.reshape(B, N, K),
            mask_i,
            mask_ij)
